# baseline (speedup 1.0000x reference)
import functools

import jax
import jax.numpy as jnp
from jax import lax
from jax.experimental import pallas as pl
from jax.experimental.pallas import tpu as pltpu
from jax.experimental.pallas import tpu_sc as plsc

NT = 4
NBIN = 400
HALO = 16


@functools.lru_cache(maxsize=None)
def _build_sc_hist(L: int, D: int):
    assert D == 16, "kernel assumes D == SC lane width (16)"
    NW = 32
    C = L // NW
    B = 2048
    NBLK = C // B
    assert C % B == 0 and L % NW == 0
    ACC = NT * NBIN * D

    mesh = plsc.VectorSubcoreMesh(core_axis_name="c", subcore_axis_name="s")

    @functools.partial(
        pl.kernel,
        mesh=mesh,
        out_type=jax.ShapeDtypeStruct((NW, ACC), jnp.float32),
        scratch_types=[
            pltpu.VMEM((ACC,), jnp.float32),
            pltpu.VMEM(((B + HALO) * D,), jnp.float32),
            pltpu.VMEM((B + HALO,), jnp.int32),
            pltpu.VMEM((2 * HALO * D,), jnp.float32),
            pltpu.VMEM((2 * HALO,), jnp.int32),
        ],
    )
    def sc_hist(seq_hbm, emb_hbm, out_hbm, acc, embv, seqv, temb, tseq):
        wid = lax.axis_index("s") * 2 + lax.axis_index("c")

        zero = jnp.zeros((D,), jnp.float32)

        def zero_body(j, carry):
            acc[pl.ds(pl.multiple_of(j * D, D), D)] = zero
            return carry

        lax.fori_loop(0, ACC // D, zero_body, None)

        wbase = wid * C

        def blk_body(b, carry):
            base = wbase + b * B
            pltpu.sync_copy(seq_hbm.at[pl.ds(pl.multiple_of(base, B), B + HALO)],
                            seqv)
            pltpu.sync_copy(
                emb_hbm.at[pl.ds(pl.multiple_of(base * D, B * D), (B + HALO) * D)],
                embv)

            @plsc.parallel_loop(0, B // 16, unroll=2)
            def grp_body(g):
                i0 = g * 16
                sA = seqv[pl.ds(pl.multiple_of(i0, 16), 16)]
                rows = [embv[pl.ds(pl.multiple_of((i0 + j) * D, D), D)]
                        for j in range(16 + NT)]
                for t in range(NT):
                    sB = seqv[pl.ds(i0 + t + 1, 16)]
                    offv = (sA * 20 + sB + t * NBIN) * D
                    for j in range(16):
                        off = pl.multiple_of(offv[j], D)
                        plsc.addupdate(acc.at[pl.ds(off, D)],
                                       rows[j] + rows[j + t + 1])

            return carry

        lax.fori_loop(0, NBLK, blk_body, None)

        @pl.when(wid == NW - 1)
        def _tail():
            tbase = L - HALO
            pltpu.sync_copy(seq_hbm.at[pl.ds(tbase, 2 * HALO)], tseq)
            pltpu.sync_copy(emb_hbm.at[pl.ds(tbase * D, 2 * HALO * D)], temb)
            sT = tseq[pl.ds(0, 16)]
            for t in range(NT):
                for m in range(t + 1):
                    li = HALO - 1 - m
                    sa = sT[li]
                    row = temb[pl.ds(li * D, D)]
                    off = (t * NBIN + sa * 20) * D
                    plsc.addupdate(acc.at[pl.ds(pl.multiple_of(off, D), D)],
                                   -row)

        pltpu.sync_copy(acc, out_hbm.at[wid])

    return sc_hist


def kernel(query_seq, emb, k):
    L = query_seq.shape[0]
    D = emb.shape[-1]
    seq_pad = jnp.concatenate(
        [query_seq.astype(jnp.int32), jnp.zeros((HALO,), jnp.int32)])
    emb_pad = jnp.concatenate(
        [emb, jnp.zeros((HALO, D), emb.dtype)]).reshape(-1)
    partials = _build_sc_hist(L, D)(seq_pad, emb_pad)
    hist = partials.sum(axis=0).reshape(NT, NBIN, D)
    t = jnp.arange(NT)
    n = (L - t - 1).astype(jnp.float32)
    gate = (t <= k).astype(jnp.float32)
    out = hist * (0.5 * gate / n)[:, None, None]
    return out.reshape(NT, 20, 20, D)


# speedup vs baseline: 1.5126x; 1.5126x over previous
import functools

import jax
import jax.numpy as jnp
from jax import lax
from jax.experimental import pallas as pl
from jax.experimental.pallas import tpu as pltpu
from jax.experimental.pallas import tpu_sc as plsc

NT = 4
NBIN = 400
HALO = 16


@functools.lru_cache(maxsize=None)
def _build_sc_hist(L: int, D: int):
    assert D == 16, "kernel assumes D == SC lane width (16)"
    NW = 32
    C = L // NW
    B = 2048
    NBLK = C // B
    assert C % B == 0 and L % NW == 0 and B % 16 == 0
    ACC = NT * NBIN * D

    mesh = plsc.VectorSubcoreMesh(core_axis_name="c", subcore_axis_name="s")

    @functools.partial(
        pl.kernel,
        mesh=mesh,
        out_type=jax.ShapeDtypeStruct((NW, ACC), jnp.float32),
        scratch_types=[
            pltpu.VMEM((ACC,), jnp.float32),
            pltpu.VMEM(((B + HALO) * D,), jnp.float32),
            pltpu.VMEM((B + HALO,), jnp.int32),
            pltpu.VMEM((2 * HALO * D,), jnp.float32),
            pltpu.VMEM((2 * HALO,), jnp.int32),
        ],
    )
    def sc_hist(seq_hbm, emb_hbm, out_hbm, acc, embv, seqv, temb, tseq):
        wid = lax.axis_index("s") * 2 + lax.axis_index("c")

        zero = jnp.zeros((D,), jnp.float32)

        def zero_body(j, carry):
            acc[pl.ds(pl.multiple_of(j * D, D), D)] = zero
            return carry

        lax.fori_loop(0, ACC // D, zero_body, None)

        wbase = wid * C

        def run_groups(ngroups):
            @plsc.parallel_loop(0, ngroups, unroll=2)
            def grp_body(g):
                i0 = g * 16
                sA = seqv[pl.ds(pl.multiple_of(i0, 16), 16)]
                rows = [embv[pl.ds(pl.multiple_of((i0 + j) * D, D), D)]
                        for j in range(16 + NT)]
                for t in range(NT):
                    sB = seqv[pl.ds(i0 + t + 1, 16)]
                    offv = (sA * 20 + sB + t * NBIN) * D
                    for j in range(16):
                        off = pl.multiple_of(offv[j], D)
                        plsc.addupdate(acc.at[pl.ds(off, D)],
                                       rows[j] + rows[j + t + 1])

        def blk_body(b, carry):
            base = wbase + b * B
            pltpu.sync_copy(seq_hbm.at[pl.ds(pl.multiple_of(base, B), B + HALO)],
                            seqv)
            pltpu.sync_copy(
                emb_hbm.at[pl.ds(pl.multiple_of(base * D, B * D), (B + HALO) * D)],
                embv)
            run_groups(B // 16)
            return carry

        lax.fori_loop(0, NBLK - 1, blk_body, None)
        last = wbase + (NBLK - 1) * B

        @pl.when(wid < NW - 1)
        def _normal_last():
            pltpu.sync_copy(seq_hbm.at[pl.ds(pl.multiple_of(last, B), B + HALO)],
                            seqv)
            pltpu.sync_copy(
                emb_hbm.at[pl.ds(pl.multiple_of(last * D, B * D), (B + HALO) * D)],
                embv)
            run_groups(B // 16)

        @pl.when(wid == NW - 1)
        def _edge_last():
            pltpu.sync_copy(seq_hbm.at[pl.ds(pl.multiple_of(last, B), B)],
                            seqv.at[pl.ds(0, B)])
            pltpu.sync_copy(
                emb_hbm.at[pl.ds(pl.multiple_of(last * D, B * D), B * D)],
                embv.at[pl.ds(0, B * D)])
            run_groups(B // 16 - 1)
            tbase = L - 2 * HALO
            pltpu.sync_copy(seq_hbm.at[pl.ds(tbase, 2 * HALO)], tseq)
            pltpu.sync_copy(emb_hbm.at[pl.ds(tbase * D, 2 * HALO * D)], temb)
            sT = tseq[pl.ds(HALO, 16)]
            for t in range(NT):
                for li in range(HALO, 2 * HALO - t - 1):
                    sa = sT[li - HALO]
                    sb = sT[li + t + 1 - HALO]
                    off = (t * NBIN + sa * 20 + sb) * D
                    row = temb[pl.ds(li * D, D)] + temb[pl.ds((li + t + 1) * D, D)]
                    plsc.addupdate(acc.at[pl.ds(pl.multiple_of(off, D), D)],
                                   row)

        pltpu.sync_copy(acc, out_hbm.at[wid])

    return sc_hist


def kernel(query_seq, emb, k):
    L = query_seq.shape[0]
    D = emb.shape[-1]
    seq32 = query_seq.astype(jnp.int32)
    emb_flat = emb.reshape(-1)
    partials = _build_sc_hist(L, D)(seq32, emb_flat)
    hist = partials.sum(axis=0).reshape(NT, NBIN, D)
    t = jnp.arange(NT)
    n = (L - t - 1).astype(jnp.float32)
    gate = (t <= k).astype(jnp.float32)
    out = hist * (0.5 * gate / n)[:, None, None]
    return out.reshape(NT, 20, 20, D)


# speedup vs baseline: 1.5807x; 1.0450x over previous
import functools

import jax
import jax.numpy as jnp
from jax import lax
from jax.experimental import pallas as pl
from jax.experimental.pallas import tpu as pltpu
from jax.experimental.pallas import tpu_sc as plsc

NT = 4
NBIN = 400
HALO = 16


@functools.lru_cache(maxsize=None)
def _build_sc_hist(L: int, D: int):
    assert D == 16, "kernel assumes D == SC lane width (16)"
    NW = 32
    C = L // NW
    B = 2048
    NBLK = C // B
    assert C % B == 0 and L % NW == 0 and B % 16 == 0
    ACC = NT * NBIN * D

    mesh = plsc.VectorSubcoreMesh(core_axis_name="c", subcore_axis_name="s")

    @functools.partial(
        pl.kernel,
        mesh=mesh,
        compiler_params=pltpu.CompilerParams(use_tc_tiling_on_sc=False),
        out_type=jax.ShapeDtypeStruct((NW, ACC), jnp.float32),
        scratch_types=[
            pltpu.VMEM((ACC,), jnp.float32),
            pltpu.VMEM((B + HALO, D), jnp.float32),
            pltpu.VMEM((B + HALO,), jnp.int32),
            pltpu.VMEM((2 * HALO, D), jnp.float32),
            pltpu.VMEM((2 * HALO,), jnp.int32),
        ],
    )
    def sc_hist(seq_hbm, emb_hbm, out_hbm, acc, embv, seqv, temb, tseq):
        wid = lax.axis_index("s") * 2 + lax.axis_index("c")

        zero = jnp.zeros((D,), jnp.float32)

        def zero_body(j, carry):
            acc[pl.ds(pl.multiple_of(j * D, D), D)] = zero
            return carry

        lax.fori_loop(0, ACC // D, zero_body, None)

        wbase = wid * C

        def run_groups(ngroups):
            @plsc.parallel_loop(0, ngroups, unroll=2)
            def grp_body(g):
                i0 = g * 16
                sA = seqv[pl.ds(pl.multiple_of(i0, 16), 16)]
                rows = [embv[i0 + j] for j in range(16 + NT)]
                for t in range(NT):
                    sB = seqv[pl.ds(i0 + t + 1, 16)]
                    offv = (sA * 20 + sB + t * NBIN) * D
                    for j in range(16):
                        off = pl.multiple_of(offv[j], D)
                        plsc.addupdate(acc.at[pl.ds(off, D)],
                                       rows[j] + rows[j + t + 1])

        def blk_body(b, carry):
            base = wbase + b * B
            pltpu.sync_copy(seq_hbm.at[pl.ds(pl.multiple_of(base, B), B + HALO)],
                            seqv)
            pltpu.sync_copy(emb_hbm.at[pl.ds(pl.multiple_of(base, B), B + HALO)],
                            embv)
            run_groups(B // 16)
            return carry

        lax.fori_loop(0, NBLK - 1, blk_body, None)
        last = wbase + (NBLK - 1) * B

        @pl.when(wid < NW - 1)
        def _normal_last():
            pltpu.sync_copy(seq_hbm.at[pl.ds(pl.multiple_of(last, B), B + HALO)],
                            seqv)
            pltpu.sync_copy(emb_hbm.at[pl.ds(pl.multiple_of(last, B), B + HALO)],
                            embv)
            run_groups(B // 16)

        @pl.when(wid == NW - 1)
        def _edge_last():
            pltpu.sync_copy(seq_hbm.at[pl.ds(pl.multiple_of(last, B), B)],
                            seqv.at[pl.ds(0, B)])
            pltpu.sync_copy(emb_hbm.at[pl.ds(pl.multiple_of(last, B), B)],
                            embv.at[pl.ds(0, B)])
            run_groups(B // 16 - 1)
            tbase = L - 2 * HALO
            pltpu.sync_copy(seq_hbm.at[pl.ds(tbase, 2 * HALO)], tseq)
            pltpu.sync_copy(emb_hbm.at[pl.ds(tbase, 2 * HALO)], temb)
            sT = tseq[pl.ds(HALO, 16)]
            for t in range(NT):
                for li in range(HALO, 2 * HALO - t - 1):
                    sa = sT[li - HALO]
                    sb = sT[li + t + 1 - HALO]
                    off = (t * NBIN + sa * 20 + sb) * D
                    row = temb[li] + temb[li + t + 1]
                    plsc.addupdate(acc.at[pl.ds(pl.multiple_of(off, D), D)],
                                   row)

        pltpu.sync_copy(acc, out_hbm.at[wid])

    return sc_hist


def kernel(query_seq, emb, k):
    L = query_seq.shape[0]
    D = emb.shape[-1]
    seq32 = query_seq.astype(jnp.int32)
    partials = _build_sc_hist(L, D)(seq32, emb)
    hist = partials.sum(axis=0).reshape(NT, NBIN, D)
    t = jnp.arange(NT)
    n = (L - t - 1).astype(jnp.float32)
    gate = (t <= k).astype(jnp.float32)
    out = hist * (0.5 * gate / n)[:, None, None]
    return out.reshape(NT, 20, 20, D)


# speedup vs baseline: 2.9498x; 1.8661x over previous
import functools

import jax
import jax.numpy as jnp
from jax import lax
from jax.experimental import pallas as pl
from jax.experimental.pallas import tpu as pltpu
from jax.experimental.pallas import tpu_sc as plsc

NT = 4
NBIN = 400
HALO = 16


@functools.lru_cache(maxsize=None)
def _build_sc_hist(L: int, D: int):
    assert D == 16, "kernel assumes D == SC lane width (16)"
    NW = 32
    C = L // NW
    B = 2048
    NBLK = C // B
    assert C % B == 0 and L % NW == 0 and B % 16 == 0
    ACC = NT * NBIN * D

    mesh = plsc.VectorSubcoreMesh(core_axis_name="c", subcore_axis_name="s")

    @functools.partial(
        pl.kernel,
        mesh=mesh,
        compiler_params=pltpu.CompilerParams(use_tc_tiling_on_sc=False,
                                             needs_layout_passes=False),
        out_type=jax.ShapeDtypeStruct((NW, ACC), jnp.float32),
        scratch_types=[
            pltpu.VMEM((ACC,), jnp.float32),
            pltpu.VMEM((D, B + HALO), jnp.float32),
            pltpu.VMEM((B + HALO,), jnp.int32),
            pltpu.VMEM((D, 2 * HALO), jnp.float32),
            pltpu.VMEM((2 * HALO,), jnp.int32),
        ],
    )
    def sc_hist(seq_hbm, embt_hbm, out_hbm, acc, embv, seqv, temb, tseq):
        wid = lax.axis_index("s") * 2 + lax.axis_index("c")
        lane = lax.iota(jnp.int32, 16)

        zero = jnp.zeros((D,), jnp.float32)

        def zero_body(j, carry):
            acc[pl.ds(pl.multiple_of(j * D, D), D)] = zero
            return carry

        lax.fori_loop(0, ACC // D, zero_body, None)

        wbase = wid * C

        def row(buf, p):
            return plsc.load_gather(buf, [lane, jnp.broadcast_to(p, (16,))])

        def run_groups(ngroups):
            @plsc.parallel_loop(0, ngroups, unroll=2)
            def grp_body(g):
                i0 = g * 16
                sA = seqv[pl.ds(pl.multiple_of(i0, 16), 16)]
                rows = [row(embv, i0 + j) for j in range(16 + NT)]
                for t in range(NT):
                    sB = seqv[pl.ds(i0 + t + 1, 16)]
                    offv = (sA * 20 + sB + t * NBIN) * D
                    for j in range(16):
                        off = pl.multiple_of(offv[j], D)
                        plsc.addupdate(acc.at[pl.ds(off, D)],
                                       rows[j] + rows[j + t + 1])

        def blk_body(b, carry):
            base = wbase + b * B
            pltpu.sync_copy(seq_hbm.at[pl.ds(pl.multiple_of(base, B), B + HALO)],
                            seqv)
            pltpu.sync_copy(embt_hbm.at[:, pl.ds(pl.multiple_of(base, B), B + HALO)],
                            embv)
            run_groups(B // 16)
            return carry

        lax.fori_loop(0, NBLK - 1, blk_body, None)
        last = wbase + (NBLK - 1) * B

        @pl.when(wid < NW - 1)
        def _normal_last():
            pltpu.sync_copy(seq_hbm.at[pl.ds(pl.multiple_of(last, B), B + HALO)],
                            seqv)
            pltpu.sync_copy(embt_hbm.at[:, pl.ds(pl.multiple_of(last, B), B + HALO)],
                            embv)
            run_groups(B // 16)

        @pl.when(wid == NW - 1)
        def _edge_last():
            pltpu.sync_copy(seq_hbm.at[pl.ds(pl.multiple_of(last, B), B)],
                            seqv.at[pl.ds(0, B)])
            pltpu.sync_copy(embt_hbm.at[:, pl.ds(pl.multiple_of(last, B), B)],
                            embv.at[:, pl.ds(0, B)])
            run_groups(B // 16 - 1)
            tbase = L - 2 * HALO
            pltpu.sync_copy(seq_hbm.at[pl.ds(tbase, 2 * HALO)], tseq)
            pltpu.sync_copy(embt_hbm.at[:, pl.ds(tbase, 2 * HALO)], temb)
            sT = tseq[pl.ds(HALO, 16)]
            for t in range(NT):
                for li in range(HALO, 2 * HALO - t - 1):
                    sa = sT[li - HALO]
                    sb = sT[li + t + 1 - HALO]
                    off = (t * NBIN + sa * 20 + sb) * D
                    v = row(temb, li) + row(temb, li + t + 1)
                    plsc.addupdate(acc.at[pl.ds(pl.multiple_of(off, D), D)],
                                   v)

        pltpu.sync_copy(acc, out_hbm.at[wid])

    return sc_hist


def kernel(query_seq, emb, k):
    L = query_seq.shape[0]
    D = emb.shape[-1]
    seq32 = query_seq.astype(jnp.int32)
    embt = emb.T
    partials = _build_sc_hist(L, D)(seq32, embt)
    hist = partials.sum(axis=0).reshape(NT, NBIN, D)
    t = jnp.arange(NT)
    n = (L - t - 1).astype(jnp.float32)
    gate = (t <= k).astype(jnp.float32)
    out = hist * (0.5 * gate / n)[:, None, None]
    return out.reshape(NT, 20, 20, D)
